# Initial kernel scaffold; baseline (speedup 1.0000x reference)
import jax
import jax.numpy as jnp
from jax.experimental import pallas as pl


def kernel(xa, W1, b1, W2, b2):
    raise NotImplementedError("write your pallas kernel here")


# speedup vs baseline: 4.0670x; 4.0670x over previous
import jax
import jax.numpy as jnp
from jax.experimental import pallas as pl

_B, _C, _BN, _H = 16, 128, 64, 32


def _edge_kernel(xa_ref, w1a_ref, w1b_ref, b1_ref, w2_ref, b2_ref, out_ref):
    x = xa_ref[0]
    a_t = jax.lax.dot_general(
        w1a_ref[...], x, dimension_numbers=(((0,), (1,)), ((), ())),
        preferred_element_type=jnp.float32,
    ) + b1_ref[...]
    b_t = jax.lax.dot_general(
        w1b_ref[...], x, dimension_numbers=(((0,), (1,)), ((), ())),
        preferred_element_type=jnp.float32,
    )
    t = jnp.maximum(a_t[:, :, None] + b_t[:, None, :], 0.0)
    logits = jnp.sum(t * w2_ref[...][:, :, None], axis=0) + b2_ref[0, 0]
    out_ref[0] = jax.nn.sigmoid(logits)


def kernel(xa, W1, b1, W2, b2):
    B, C, BN = xa.shape
    H = W1.shape[1]
    w1a = W1[:BN]
    w1b = W1[BN:]
    b1c = b1.reshape(H, 1)
    w2c = W2.reshape(H, 1)
    b2s = b2.reshape(1, 1)
    return pl.pallas_call(
        _edge_kernel,
        grid=(B,),
        in_specs=[
            pl.BlockSpec((1, C, BN), lambda b: (b, 0, 0)),
            pl.BlockSpec((BN, H), lambda b: (0, 0)),
            pl.BlockSpec((BN, H), lambda b: (0, 0)),
            pl.BlockSpec((H, 1), lambda b: (0, 0)),
            pl.BlockSpec((H, 1), lambda b: (0, 0)),
            pl.BlockSpec((1, 1), lambda b: (0, 0)),
        ],
        out_specs=pl.BlockSpec((1, C, C), lambda b: (b, 0, 0)),
        out_shape=jax.ShapeDtypeStruct((B, C, C), jnp.float32),
    )(xa, w1a, w1b, b1c, w2c, b2s)
